# baseline (speedup 1.0000x reference)
import functools

import jax
import jax.numpy as jnp
from jax import lax
from jax.experimental import pallas as pl
from jax.experimental.pallas import tpu as pltpu
from jax.experimental.pallas import tpu_sc as plsc

_V = 10000
_VPAD = 10240
_D = 128
_H = 512
_W = 512
_P = _H * _W

_NC = 2
_NS = 16
_NW = _NC * _NS
_CPW = 8
_NCG = _D // _CPW
_NPH = _NW // _NCG
_PPW = _P // _NPH
_CHUNK = 2048
_NCHUNK = _PPW // _CHUNK
_ROWS = _CHUNK // _W
_LANES = 16
_UNROLL = 16


def _transpose_body(x_ref, o_ref):
    o_ref[...] = jnp.pad(x_ref[...], ((0, _VPAD - _V), (0, 0))).T


def _table_transpose(table):
    return pl.pallas_call(
        _transpose_body,
        out_shape=jax.ShapeDtypeStruct((_D, _VPAD), jnp.float32),
    )(table)


def _gather_body(table_t_hbm, idx_hbm, out_hbm, rows_v, idx_v, out_v,
                 sem_idx0, sem_idx1, sem_out0, sem_out1):
    wid = lax.axis_index("s") * _NC + lax.axis_index("c")
    c0 = (wid // _NPH) * _CPW
    p0 = (wid % _NPH) * _PPW
    r0 = (wid % _NPH) * (_PPW // _W)
    sem_idx = (sem_idx0, sem_idx1)
    sem_out = (sem_out0, sem_out1)

    def idx_copy(t, tb):
        return pltpu.make_async_copy(
            idx_hbm.at[pl.ds(r0 + t * 2 * _ROWS, 2 * _ROWS)],
            idx_v.at[tb], sem_idx[tb])

    def out_copy(g, b, c):
        return pltpu.make_async_copy(
            out_v.at[b * _CPW + c],
            out_hbm.at[c0 + c].at[pl.ds(r0 + g * _ROWS, _ROWS)], sem_out[b])

    def gather_chunk(tb, b):
        @plsc.parallel_loop(0, _CHUNK // _LANES, unroll=_UNROLL)
        def vec_body(i):
            p = i * _LANES
            h = i // (_W // _LANES)
            w = p % _W
            idx16 = idx_v[tb, b * _ROWS + h, pl.ds(w, _LANES)]
            for c in range(_CPW):
                cvec = jnp.full((_LANES,), c, jnp.int32)
                out_v[b * _CPW + c, h, pl.ds(w, _LANES)] = (
                    plsc.load_gather(rows_v, [cvec, idx16]))

    idx_copy(0, 0).start()
    idx_copy(1, 1).start()
    pltpu.sync_copy(table_t_hbm.at[pl.ds(c0, _CPW)], rows_v)

    def quad_body(t2, carry):
        for tb in range(2):
            t = 2 * t2 + tb
            idx_copy(t, tb).wait()
            for b in range(2):
                g = 2 * t + b

                if tb == 0:
                    @pl.when(t2 > 0)
                    def _wait_out():
                        for c in range(_CPW):
                            out_copy(g - 2, b, c).wait()
                else:
                    for c in range(_CPW):
                        out_copy(g - 2, b, c).wait()

                gather_chunk(tb, b)

                for c in range(_CPW):
                    out_copy(g, b, c).start()

            @pl.when(t + 2 < _NCHUNK // 2)
            def _next_idx():
                idx_copy(t + 2, tb).start()
        return carry

    lax.fori_loop(0, _NCHUNK // 4, quad_body, 0)
    for g in (_NCHUNK - 2, _NCHUNK - 1):
        for c in range(_CPW):
            out_copy(g, g % 2, c).wait()


_gather_call = functools.partial(
    pl.kernel,
    out_type=jax.ShapeDtypeStruct((_D, _H, _W), jnp.float32),
    mesh=plsc.VectorSubcoreMesh(core_axis_name="c", subcore_axis_name="s"),
    scratch_types=[
        pltpu.VMEM((_CPW, _VPAD), jnp.float32),
        pltpu.VMEM((2, 2 * _ROWS, _W), jnp.int32),
        pltpu.VMEM((2 * _CPW, _ROWS, _W), jnp.float32),
        pltpu.SemaphoreType.DMA,
        pltpu.SemaphoreType.DMA,
        pltpu.SemaphoreType.DMA,
        pltpu.SemaphoreType.DMA,
    ],
    compiler_params=pltpu.CompilerParams(
        needs_layout_passes=False, use_tc_tiling_on_sc=True),
)(_gather_body)


def kernel(graph_nodes, spx_image):
    table_t = _table_transpose(graph_nodes)
    return _gather_call(table_t, spx_image.astype(jnp.int32))


# speedup vs baseline: 1.3576x; 1.3576x over previous
import functools

import jax
import jax.numpy as jnp
from jax import lax
from jax.experimental import pallas as pl
from jax.experimental.pallas import tpu as pltpu
from jax.experimental.pallas import tpu_sc as plsc

_V = 10000
_VPAD = 10240
_D = 128
_H = 512
_W = 512
_P = _H * _W

_NC = 2
_NS = 16
_NW = _NC * _NS
_CPW = 8
_NCG = _D // _CPW
_NPH = _NW // _NCG
_PPW = _P // _NPH
_CHUNK = 2048
_NCHUNK = _PPW // _CHUNK
_ROWS = _CHUNK // _W
_LANES = 16
_UNROLL = 8


def _transpose_body(x_ref, o_ref):
    o_ref[...] = jnp.pad(x_ref[...], ((0, _VPAD - _V), (0, 0))).T


def _table_transpose(table):
    return pl.pallas_call(
        _transpose_body,
        out_shape=jax.ShapeDtypeStruct((_D, _VPAD), jnp.float32),
    )(table)


def _gather_body(table_t_hbm, idx_hbm, out_hbm, rows_v, idx_v, out_v,
                 sem_idx0, sem_idx1, sem_out0, sem_out1):
    wid = lax.axis_index("s") * _NC + lax.axis_index("c")
    c0 = (wid // _NPH) * _CPW
    p0 = (wid % _NPH) * _PPW
    r0 = (wid % _NPH) * (_PPW // _W)
    sem_idx = (sem_idx0, sem_idx1)
    sem_out = (sem_out0, sem_out1)

    def idx_copy(t, tb):
        return pltpu.make_async_copy(
            idx_hbm.at[pl.ds(r0 + t * 2 * _ROWS, 2 * _ROWS)],
            idx_v.at[tb], sem_idx[tb])

    def out_copy(t, b):
        return pltpu.make_async_copy(
            out_v.at[b],
            out_hbm.at[pl.ds(c0, _CPW), pl.ds(r0 + t * 8, 8),
                       pl.ds(b * (_W // 2), _W // 2)],
            sem_out[b])

    def gather_chunk(tb, b):
        @plsc.parallel_loop(0, _CHUNK // _LANES, unroll=_UNROLL)
        def vec_body(i):
            h = i // (_W // 2 // _LANES)
            w = (i % (_W // 2 // _LANES)) * _LANES
            idx16 = idx_v[tb, h, pl.ds(b * (_W // 2) + w, _LANES)]
            for c in range(_CPW):
                cvec = jnp.full((_LANES,), c, jnp.int32)
                out_v[b, c, h, pl.ds(w, _LANES)] = (
                    plsc.load_gather(rows_v, [cvec, idx16]))

    idx_copy(0, 0).start()
    idx_copy(1, 1).start()
    pltpu.sync_copy(table_t_hbm.at[pl.ds(c0, _CPW)], rows_v)

    def quad_body(t2, carry):
        for tb in range(2):
            t = 2 * t2 + tb
            idx_copy(t, tb).wait()
            for b in range(2):
                if tb == 0:
                    @pl.when(t2 > 0)
                    def _wait_out():
                        out_copy(t - 1, b).wait()
                else:
                    out_copy(t - 1, b).wait()

                gather_chunk(tb, b)
                out_copy(t, b).start()

            @pl.when(t + 2 < _NCHUNK // 2)
            def _next_idx():
                idx_copy(t + 2, tb).start()
        return carry

    lax.fori_loop(0, _NCHUNK // 4, quad_body, 0)
    for b in range(2):
        out_copy(_NCHUNK // 2 - 1, b).wait()


_gather_call = functools.partial(
    pl.kernel,
    out_type=jax.ShapeDtypeStruct((_D, _H, _W), jnp.float32),
    mesh=plsc.VectorSubcoreMesh(core_axis_name="c", subcore_axis_name="s"),
    scratch_types=[
        pltpu.VMEM((_CPW, _VPAD), jnp.float32),
        pltpu.VMEM((2, 2 * _ROWS, _W), jnp.int32),
        pltpu.VMEM((2, _CPW, 8, _W // 2), jnp.float32),
        pltpu.SemaphoreType.DMA,
        pltpu.SemaphoreType.DMA,
        pltpu.SemaphoreType.DMA,
        pltpu.SemaphoreType.DMA,
    ],
    compiler_params=pltpu.CompilerParams(
        needs_layout_passes=False, use_tc_tiling_on_sc=True),
)(_gather_body)


def kernel(graph_nodes, spx_image):
    table_t = _table_transpose(graph_nodes)
    return _gather_call(table_t, spx_image.astype(jnp.int32))
